# baseline (speedup 1.0000x reference)
import functools
import jax
import jax.numpy as jnp
from jax import lax
from jax.experimental import pallas as pl
from jax.experimental.pallas import tpu as pltpu
from jax.experimental.pallas import tpu_sc as plsc

_D = 2048
_B = 4
_QK_SCALE = 16 ** -0.5
_SUBSTEPS = 8
_TPR = 8
_CHUNK = _D // _TPR
_NCH = _CHUNK // 16
_F32 = jnp.float32


def _bcast(v, lane):
    return jnp.take_along_axis(v, jnp.full((16,), lane, jnp.int32), axis=0)


def _hsum(v):
    return jnp.sum(v)


def _sc_body(x_hbm, ep_hbm, ept_hbm, w_hbm, idx_hbm, out_hbm,
             ep_v, ept_v, x_v, xs_v, acc_v,
             w_v, idx_v, part_v, tot_v, zero_v, accum_sh):
    cid = lax.axis_index("c")
    sid = lax.axis_index("s")
    row_l = sid // _TPR
    p = sid % _TPR
    r = cid * 2 + row_l
    j0 = p * _CHUNK

    pltpu.sync_copy(x_hbm.at[r, pl.ds(j0, _CHUNK)], x_v)
    pltpu.sync_copy(ep_hbm.at[pl.ds(j0 * 16, _CHUNK * 16)], ep_v)
    pltpu.sync_copy(ept_hbm.at[p], ept_v)
    pltpu.sync_copy(w_hbm, w_v)
    pltpu.sync_copy(idx_hbm, idx_v)
    pltpu.sync_copy(x_v, out_hbm.at[0, r, pl.ds(j0, _CHUNK)])

    for c in range(_NCH):
        sl = pl.ds(c * 16, 16)
        xs_v[sl] = x_v[sl]
    zv = jnp.zeros((16,), _F32)
    for i in range(3, 8):
        part_v[0, pl.ds(i * 16, 16)] = zv
    for i in range(8):
        zero_v[0, pl.ds(i * 16, 16)] = zv

    wq0 = w_v[pl.ds(544, 16)]
    wk0 = w_v[pl.ds(560, 16)]
    hv = w_v[pl.ds(576, 16)]
    bqv = w_v[pl.ds(512, 16)]
    bkv = w_v[pl.ds(528, 16)]

    @pl.when(p == 0)
    def _():
        pltpu.sync_copy(zero_v, accum_sh.at[pl.ds(row_l * 4, 1)])
        pltpu.sync_copy(zero_v, accum_sh.at[pl.ds(row_l * 4 + 1, 1)])

    plsc.subcore_barrier()

    def do_stage(stage_i, par):
        z = jnp.zeros((16,), _F32)
        ue, ssv, sxv = z, z, z
        for c in range(_NCH):
            xc = xs_v[pl.ds(c * 16, 16)]
            ssv = ssv + xc * xc
            sxv = sxv + xc
            for l in range(16):
                bx = _bcast(xc, l)
                ue = ue + bx * ep_v[pl.ds((c * 16 + l) * 16, 16)]
        part_v[0, pl.ds(0, 16)] = ue
        part_v[0, pl.ds(16, 16)] = ssv
        part_v[0, pl.ds(32, 16)] = sxv

        pltpu.sync_copy(part_v, accum_sh.at[idx_v.at[row_l * 4 + par]],
                        add=True)
        plsc.subcore_barrier()
        pltpu.sync_copy(accum_sh.at[row_l * 4 + par], tot_v)

        @pl.when(p == 0)
        def _():
            pltpu.sync_copy(
                zero_v, accum_sh.at[pl.ds(row_l * 4 + (par + 2) % 4, 1)])

        uet = tot_v[pl.ds(0, 16)]
        ss = _hsum(tot_v[pl.ds(16, 16)])
        sx = _hsum(tot_v[pl.ds(32, 16)])
        uk = sx * bkv
        for e in range(16):
            uk = uk + _bcast(uet, e) * w_v[pl.ds(e * 16, 16)]
        m = wk0 * ss + uk
        a = _hsum(wq0 * m)
        mq = jnp.zeros((16,), _F32)
        for d in range(16):
            mq = mq + _bcast(m, d) * w_v[pl.ds(256 + d * 16, 16)]
        bqm = _hsum(bqv * m)
        g = _QK_SCALE * (a * ss + _hsum(uet * mq) + sx * bqm)
        sbqm = _QK_SCALE * bqm
        mqb = [_QK_SCALE * _bcast(mq, e) for e in range(16)]
        sa = _QK_SCALE * a

        for c in range(_NCH):
            sl = pl.ds(c * 16, 16)
            xsc = xs_v[sl]
            pf = mqb[0] * ept_v[0, sl]
            for d in range(1, 16):
                pf = pf + mqb[d] * ept_v[d, sl]
            fit = sa * xsc + pf + sbqm
            kc = xsc * (fit - g)
            if stage_i == 0:
                acc_v[sl] = kc
                xs_v[sl] = x_v[sl] + (0.5 * hv) * kc
            elif stage_i == 1:
                acc_v[sl] = acc_v[sl] + 2.0 * kc
                xs_v[sl] = x_v[sl] + (0.5 * hv) * kc
            elif stage_i == 2:
                acc_v[sl] = acc_v[sl] + 2.0 * kc
                xs_v[sl] = x_v[sl] + hv * kc
            else:
                xn = x_v[sl] + (hv * (1.0 / 6.0)) * (acc_v[sl] + kc)
                x_v[sl] = xn
                xs_v[sl] = xn

    def step(_, carry):
        do_stage(0, 0)
        do_stage(1, 1)
        do_stage(2, 2)
        do_stage(3, 3)
        return carry

    lax.fori_loop(0, _SUBSTEPS, step, 0)

    pltpu.sync_copy(x_v, out_hbm.at[1, r, pl.ds(j0, _CHUNK)])


def kernel(t, x, embed_table, wq, bq, wk, bk):
    B, D = x.shape
    ep2 = jnp.concatenate(
        [jnp.zeros((D, 1), _F32), embed_table[1:D + 1]], axis=1)
    ep = ep2.reshape(-1)
    ept8 = jnp.stack([ep2.T[:, i * _CHUNK:(i + 1) * _CHUNK]
                      for i in range(_TPR)], axis=0)
    h = (t[1] - t[0]) / _SUBSTEPS
    wcat = jnp.concatenate([
        wk.reshape(-1), wq.T.reshape(-1), bq, bk, wq[0], wk[0],
        jnp.broadcast_to(h, (16,)),
        jnp.zeros((48,), _F32)])
    idx_all = jnp.arange(8, dtype=jnp.int32).reshape(8, 1)

    mesh = plsc.VectorSubcoreMesh(core_axis_name="c", subcore_axis_name="s",
                                  num_cores=2, num_subcores=16)
    run = functools.partial(
        pl.kernel,
        out_type=jax.ShapeDtypeStruct((2, B, D), _F32),
        mesh=mesh,
        compiler_params=pltpu.CompilerParams(needs_layout_passes=False),
        scratch_types=[
            pltpu.VMEM((_CHUNK * 16,), _F32),
            pltpu.VMEM((16, _CHUNK), _F32),
            pltpu.VMEM((_CHUNK,), _F32),
            pltpu.VMEM((_CHUNK,), _F32),
            pltpu.VMEM((_CHUNK,), _F32),
            pltpu.VMEM((640,), _F32),
            pltpu.VMEM((8, 1), jnp.int32),
            pltpu.VMEM((1, 128), _F32),
            pltpu.VMEM((128,), _F32),
            pltpu.VMEM((1, 128), _F32),
            pltpu.VMEM_SHARED((8, 128), _F32),
        ],
    )(_sc_body)
    return run(x, ep, ept8, wcat, idx_all)


# speedup vs baseline: 1.0873x; 1.0873x over previous
import functools
import jax
import jax.numpy as jnp
from jax import lax
from jax.experimental import pallas as pl
from jax.experimental.pallas import tpu as pltpu
from jax.experimental.pallas import tpu_sc as plsc

_D = 2048
_B = 4
_QK_SCALE = 16 ** -0.5
_SUBSTEPS = 8
_TPR = 8
_CHUNK = _D // _TPR
_NCH = _CHUNK // 16
_F32 = jnp.float32


def _bcast(v, lane):
    return jnp.take_along_axis(v, jnp.full((16,), lane, jnp.int32), axis=0)


def _hsum(v):
    return jnp.sum(v)


def _sc_body(x_hbm, ep_hbm, ept_hbm, w_hbm, out_hbm,
             ep_v, ept_v, x_v, xs_v, acc_v,
             w_v, part_v, all_v, accum_sh):
    cid = lax.axis_index("c")
    sid = lax.axis_index("s")
    row_l = sid // _TPR
    p = sid % _TPR
    r = cid * 2 + row_l
    j0 = p * _CHUNK

    pltpu.sync_copy(x_hbm.at[r, pl.ds(j0, _CHUNK)], x_v)
    pltpu.sync_copy(ep_hbm.at[pl.ds(j0 * 16, _CHUNK * 16)], ep_v)
    pltpu.sync_copy(ept_hbm.at[p], ept_v)
    pltpu.sync_copy(w_hbm, w_v)
    pltpu.sync_copy(x_v, out_hbm.at[0, r, pl.ds(j0, _CHUNK)])

    for c in range(_NCH):
        sl = pl.ds(c * 16, 16)
        xs_v[sl] = x_v[sl]
    zv = jnp.zeros((16,), _F32)
    for i in range(3, 8):
        part_v[pl.ds(i * 16, 16)] = zv

    wq0 = w_v[pl.ds(544, 16)]
    wk0 = w_v[pl.ds(560, 16)]
    hv = w_v[pl.ds(576, 16)]
    bqv = w_v[pl.ds(512, 16)]
    bkv = w_v[pl.ds(528, 16)]

    def do_stage(stage_i, par):
        z = jnp.zeros((16,), _F32)
        ue, ssv, sxv = z, z, z
        for c in range(_NCH):
            xc = xs_v[pl.ds(c * 16, 16)]
            ssv = ssv + xc * xc
            sxv = sxv + xc
            for l in range(16):
                bx = _bcast(xc, l)
                ue = ue + bx * ep_v[pl.ds((c * 16 + l) * 16, 16)]
        part_v[pl.ds(0, 16)] = ue
        part_v[pl.ds(16, 16)] = ssv
        part_v[pl.ds(32, 16)] = sxv

        pltpu.sync_copy(part_v, accum_sh.at[row_l, par, p])
        plsc.subcore_barrier()
        pltpu.sync_copy(accum_sh.at[row_l, par], all_v)

        uet = all_v[0, pl.ds(0, 16)]
        ssw = all_v[0, pl.ds(16, 16)]
        sxw = all_v[0, pl.ds(32, 16)]
        for q in range(1, _TPR):
            uet = uet + all_v[q, pl.ds(0, 16)]
            ssw = ssw + all_v[q, pl.ds(16, 16)]
            sxw = sxw + all_v[q, pl.ds(32, 16)]
        ss = _hsum(ssw)
        sx = _hsum(sxw)
        uk = sx * bkv
        for e in range(16):
            uk = uk + _bcast(uet, e) * w_v[pl.ds(e * 16, 16)]
        m = wk0 * ss + uk
        a = _hsum(wq0 * m)
        mq = jnp.zeros((16,), _F32)
        for d in range(16):
            mq = mq + _bcast(m, d) * w_v[pl.ds(256 + d * 16, 16)]
        bqm = _hsum(bqv * m)
        g = _QK_SCALE * (a * ss + _hsum(uet * mq) + sx * bqm)
        sbqm = _QK_SCALE * bqm
        mqb = [_QK_SCALE * _bcast(mq, e) for e in range(16)]
        sa = _QK_SCALE * a

        for c in range(_NCH):
            sl = pl.ds(c * 16, 16)
            xsc = xs_v[sl]
            pf = mqb[0] * ept_v[0, sl]
            for d in range(1, 16):
                pf = pf + mqb[d] * ept_v[d, sl]
            fit = sa * xsc + pf + sbqm
            kc = xsc * (fit - g)
            if stage_i == 0:
                acc_v[sl] = kc
                xs_v[sl] = x_v[sl] + (0.5 * hv) * kc
            elif stage_i == 1:
                acc_v[sl] = acc_v[sl] + 2.0 * kc
                xs_v[sl] = x_v[sl] + (0.5 * hv) * kc
            elif stage_i == 2:
                acc_v[sl] = acc_v[sl] + 2.0 * kc
                xs_v[sl] = x_v[sl] + hv * kc
            else:
                xn = x_v[sl] + (hv * (1.0 / 6.0)) * (acc_v[sl] + kc)
                x_v[sl] = xn
                xs_v[sl] = xn

    def step(_, carry):
        do_stage(0, 0)
        do_stage(1, 1)
        do_stage(2, 0)
        do_stage(3, 1)
        return carry

    lax.fori_loop(0, _SUBSTEPS, step, 0)

    pltpu.sync_copy(x_v, out_hbm.at[1, r, pl.ds(j0, _CHUNK)])


def kernel(t, x, embed_table, wq, bq, wk, bk):
    B, D = x.shape
    ep2 = jnp.concatenate(
        [jnp.zeros((D, 1), _F32), embed_table[1:D + 1]], axis=1)
    ep = ep2.reshape(-1)
    ept8 = jnp.stack([ep2.T[:, i * _CHUNK:(i + 1) * _CHUNK]
                      for i in range(_TPR)], axis=0)
    h = (t[1] - t[0]) / _SUBSTEPS
    wcat = jnp.concatenate([
        wk.reshape(-1), wq.T.reshape(-1), bq, bk, wq[0], wk[0],
        jnp.broadcast_to(h, (16,)),
        jnp.zeros((48,), _F32)])

    mesh = plsc.VectorSubcoreMesh(core_axis_name="c", subcore_axis_name="s",
                                  num_cores=2, num_subcores=16)
    run = functools.partial(
        pl.kernel,
        out_type=jax.ShapeDtypeStruct((2, B, D), _F32),
        mesh=mesh,
        compiler_params=pltpu.CompilerParams(needs_layout_passes=False),
        scratch_types=[
            pltpu.VMEM((_CHUNK * 16,), _F32),
            pltpu.VMEM((16, _CHUNK), _F32),
            pltpu.VMEM((_CHUNK,), _F32),
            pltpu.VMEM((_CHUNK,), _F32),
            pltpu.VMEM((_CHUNK,), _F32),
            pltpu.VMEM((640,), _F32),
            pltpu.VMEM((128,), _F32),
            pltpu.VMEM((_TPR, 128), _F32),
            pltpu.VMEM_SHARED((2, 2, _TPR, 128), _F32),
        ],
    )(_sc_body)
    return run(x, ep, ept8, wcat)
